# baseline (speedup 1.0000x reference)
import functools

import jax
import jax.numpy as jnp
from jax import lax
from jax.experimental import pallas as pl
from jax.experimental.pallas import tpu as pltpu
from jax.experimental.pallas import tpu_sc as plsc

N = 10000
E = 320000
D = 128
L = 5
B = 128
V = 5002
S = 5
MAX_DEPTH = 20

NW = 32
NPAD = 10240
NPW = NPAD // NW
NCH = 80
ECH = 128
SB = 8
NSUP = 10
TOT_SUP = NW * NSUP
EPAD = TOT_SUP * SB * ECH
SLICE = NPAD // 16
VPAD = 5120
MB = 1024
PB = 1024


def _worker_id():
    return lax.axis_index("s") * 2 + lax.axis_index("c")


def _embed_chunk_start(ci, idx_hbm, temb_hbm, aemb_hbm, demb_hbm,
                       ib, tr, ar, dr, sem):
    pltpu.sync_copy(idx_hbm.at[ci], ib)
    pltpu.async_copy(temb_hbm.at[ib.at[0]], tr, sem)
    pltpu.async_copy(aemb_hbm.at[ib.at[1]], ar, sem)
    pltpu.async_copy(demb_hbm.at[ib.at[2]], dr, sem)


def _embed_body(idx_hbm, temb_hbm, aemb_hbm, demb_hbm, out_hbm,
                ib0, ib1, tr0, tr1, ar0, ar1, dr0, dr1, sem0, sem1):
    wid = _worker_id()
    base0 = wid * NPW
    nch = NPW // NCH
    ci0 = wid * nch
    ibs = (ib0, ib1)
    trs = (tr0, tr1)
    ars = (ar0, ar1)
    drs = (dr0, dr1)
    sems = (sem0, sem1)
    for p in range(2):
        _embed_chunk_start(ci0 + p, idx_hbm, temb_hbm, aemb_hbm,
                           demb_hbm, ibs[p], trs[p], ars[p], drs[p], sems[p])

    @pl.loop(0, nch, step=2)
    def _chunk(c):
        for p in range(2):
            cc = c + p
            base = pl.multiple_of(base0 + cc * NCH, NCH)
            for r in (trs[p], ars[p], drs[p]):
                pltpu.make_async_copy(temb_hbm.at[ibs[p].at[0]], r,
                                      sems[p]).wait()

            @pl.loop(0, NCH, unroll=4)
            def _row(r):
                for j in range(D // 16):
                    sl = pl.ds(j * 16, 16)
                    trs[p][r, sl] = (trs[p][r, sl] + ars[p][r, sl]
                                     + drs[p][r, sl])

            pltpu.sync_copy(trs[p], out_hbm.at[pl.ds(base, NCH)])
            nxt = jnp.minimum(ci0 + cc + 2, ci0 + nch - 1)
            _embed_chunk_start(nxt, idx_hbm, temb_hbm, aemb_hbm, demb_hbm,
                               ibs[p], trs[p], ars[p], drs[p], sems[p])

    for p in range(2):
        for r in (trs[p], ars[p], drs[p]):
            pltpu.make_async_copy(temb_hbm.at[ibs[p].at[0]], r,
                                  sems[p]).wait()


def _embed(idx3, temb, aemb, demb):
    mesh = plsc.VectorSubcoreMesh(core_axis_name="c", subcore_axis_name="s")
    call = pl.kernel(
        _embed_body,
        out_type=jax.ShapeDtypeStruct((NPAD, D), jnp.float32),
        mesh=mesh,
        scratch_types=[
            pltpu.VMEM((3, NCH), jnp.int32),
            pltpu.VMEM((3, NCH), jnp.int32),
            pltpu.VMEM((NCH, D), jnp.float32),
            pltpu.VMEM((NCH, D), jnp.float32),
            pltpu.VMEM((NCH, D), jnp.float32),
            pltpu.VMEM((NCH, D), jnp.float32),
            pltpu.VMEM((NCH, D), jnp.float32),
            pltpu.VMEM((NCH, D), jnp.float32),
            pltpu.SemaphoreType.DMA,
            pltpu.SemaphoreType.DMA,
        ],
    )
    return call(idx3, temb, aemb, demb)


def _edge_pipeline(nsup, sup0, h_hbm, sd_hbm, sdb, isems, rows, sems, agg_sp):
    pltpu.async_copy(sd_hbm.at[sup0], sdb[0], isems[0])
    pltpu.async_copy(sd_hbm.at[sup0 + 1], sdb[1], isems[1])

    @pl.loop(0, nsup, step=2)
    def _super(so):
        for p in range(2):
            s = sup0 + so + p
            pltpu.make_async_copy(sd_hbm.at[s], sdb[p], isems[p]).wait()
            pltpu.async_copy(h_hbm.at[sdb[p].at[0, 0]], rows[0], sems[0])
            pltpu.async_copy(h_hbm.at[sdb[p].at[0, 1]], rows[1], sems[1])
            for j in range(SB):
                b = j % 2
                pltpu.make_async_copy(h_hbm.at[sdb[p].at[0, j]], rows[b],
                                      sems[b]).wait()
                pltpu.sync_copy(rows[b], agg_sp.at[sdb[p].at[1, j]],
                                add=True)
                if j < SB - 2:
                    pltpu.async_copy(h_hbm.at[sdb[p].at[0, j + 2]],
                                     rows[b], sems[b])
            nxt = jnp.minimum(s + 2, sup0 + nsup - 1)
            pltpu.async_copy(sd_hbm.at[nxt], sdb[p], isems[p])

    for p in range(2):
        pltpu.make_async_copy(sd_hbm.at[sup0], sdb[p], isems[p]).wait()


def _agg_body(h_hbm, sd_hbm, zz_hbm, out_hbm,
              sd0, sd1, rows0, rows1, agg_sp, isem0, isem1, sem0, sem1):
    cid = lax.axis_index("c")
    sid = lax.axis_index("s")
    sdb = (sd0, sd1)
    isems = (isem0, isem1)
    rows = (rows0, rows1)
    sems = (sem0, sem1)
    with jax.named_scope("agg_zero"):
        pltpu.sync_copy(zz_hbm, agg_sp.at[pl.ds(sid * SLICE, SLICE)])
        plsc.subcore_barrier()

    with jax.named_scope("agg_edges"):
        wid = sid * 2 + cid
        _edge_pipeline(NSUP, wid * NSUP, h_hbm, sd_hbm, sdb, isems,
                       rows, sems, agg_sp)

    with jax.named_scope("agg_drain"):
        plsc.subcore_barrier()
        nd = SLICE // ECH
        for k in range(nd):
            b = k % 2
            base = sid * SLICE + k * ECH
            if k >= 2:
                pltpu.make_async_copy(
                    rows[b], out_hbm.at[cid, pl.ds(base, ECH)],
                    sems[b]).wait()
            pltpu.sync_copy(agg_sp.at[pl.ds(base, ECH)], rows[b])
            pltpu.async_copy(rows[b], out_hbm.at[cid, pl.ds(base, ECH)],
                             sems[b])
        for b in range(2):
            pltpu.make_async_copy(
                rows[b], out_hbm.at[cid, pl.ds(sid * SLICE, ECH)],
                sems[b]).wait()


def _agg(h, sd, zeros_slice):
    mesh = plsc.VectorSubcoreMesh(core_axis_name="c", subcore_axis_name="s")
    call = pl.kernel(
        _agg_body,
        out_type=jax.ShapeDtypeStruct((2, NPAD, D), jnp.float32),
        mesh=mesh,
        scratch_types=[
            pltpu.VMEM((2, SB, ECH), jnp.int32),
            pltpu.VMEM((2, SB, ECH), jnp.int32),
            pltpu.VMEM((ECH, D), jnp.float32),
            pltpu.VMEM((ECH, D), jnp.float32),
            pltpu.VMEM_SHARED((NPAD, D), jnp.float32),
            pltpu.SemaphoreType.DMA,
            pltpu.SemaphoreType.DMA,
            pltpu.SemaphoreType.DMA,
            pltpu.SemaphoreType.DMA,
        ],
    )
    return call(h, sd, zeros_slice)


def _mlp_body(scale_sr, h_ref, a_ref, w1_ref, b1_ref, w2_ref, b2_ref,
              g_ref, bb_ref, o_ref, *, last):
    x = h_ref[...] * scale_sr[0] + a_ref[0] + a_ref[1]
    y = jnp.dot(x, w1_ref[...], preferred_element_type=jnp.float32)
    y = jnp.maximum(y + b1_ref[...], 0.0)
    z = jnp.dot(y, w2_ref[...], preferred_element_type=jnp.float32)
    z = (z + b2_ref[...]) * g_ref[...] + bb_ref[...]
    if not last:
        z = jnp.maximum(z, 0.0)
    o_ref[...] = z


def _mlp(last, scale, h, parts, w1, b1, w2, b2, bng, bnb):
    call = pl.pallas_call(
        functools.partial(_mlp_body, last=last),
        grid=(NPAD // MB,),
        in_specs=[
            pl.BlockSpec(memory_space=pltpu.SMEM),
            pl.BlockSpec((MB, D), lambda i: (i, 0)),
            pl.BlockSpec((2, MB, D), lambda i: (0, i, 0)),
            pl.BlockSpec((D, 2 * D), lambda i: (0, 0)),
            pl.BlockSpec((1, 2 * D), lambda i: (0, 0)),
            pl.BlockSpec((2 * D, D), lambda i: (0, 0)),
            pl.BlockSpec((1, D), lambda i: (0, 0)),
            pl.BlockSpec((1, D), lambda i: (0, 0)),
            pl.BlockSpec((1, D), lambda i: (0, 0)),
        ],
        out_specs=pl.BlockSpec((MB, D), lambda i: (i, 0)),
        out_shape=jax.ShapeDtypeStruct((NPAD, D), jnp.float32),
    )
    return call(scale, h, parts, w1, b1, w2, b2, bng, bnb)


def _pool_body(h_ref, batch_ref, o_ref):
    def step(i, acc):
        sums, cnts = acc
        hb = h_ref[pl.ds(i * PB, PB), :]
        bb = batch_ref[pl.ds(i * PB, PB), :]
        hb = jnp.where(bb < B, hb, 0.0)
        oh = (bb == lax.broadcasted_iota(jnp.int32, (1, B), 1))
        oh = oh.astype(jnp.float32)
        sums = sums + lax.dot_general(oh, hb, (((0,), (0,)), ((), ())),
                                      preferred_element_type=jnp.float32)
        cnts = cnts + jnp.sum(oh, axis=0)
        return sums, cnts

    sums, cnts = lax.fori_loop(
        0, NPAD // PB, step,
        (jnp.zeros((B, D), jnp.float32), jnp.zeros((B,), jnp.float32)))
    o_ref[...] = sums / jnp.maximum(cnts, 1.0)[:, None]


def _pool(h, batch_p):
    call = pl.pallas_call(
        _pool_body,
        out_shape=jax.ShapeDtypeStruct((B, D), jnp.float32),
    )
    return call(h, batch_p)


def _heads_body(g_ref, w_ref, b_ref, o_ref):
    o_ref[0] = jnp.dot(g_ref[...], w_ref[0],
                       preferred_element_type=jnp.float32) + b_ref[0]


def _heads(g, wp, bp):
    call = pl.pallas_call(
        _heads_body,
        grid=(S,),
        in_specs=[
            pl.BlockSpec((B, D), lambda s: (0, 0)),
            pl.BlockSpec((1, D, V), lambda s: (s, 0, 0)),
            pl.BlockSpec((1, 1, V), lambda s: (s, 0, 0)),
        ],
        out_specs=pl.BlockSpec((1, B, V), lambda s: (s, 0, 0)),
        out_shape=jax.ShapeDtypeStruct((S, B, V), jnp.float32),
    )
    return call(g, wp, bp)


def kernel(node_feat, node_depth, edge_index, batch, type_emb, attr_emb,
           depth_emb, eps, W1, b1, W2, b2, bn_g, bn_b, head_W, head_b):
    f32 = jnp.float32
    i32 = jnp.int32
    npad = NPAD - N
    idx3 = jnp.stack([
        jnp.pad(node_feat[:, 0].astype(i32), (0, npad)),
        jnp.pad(node_feat[:, 1].astype(i32), (0, npad)),
        jnp.pad(jnp.clip(node_depth, 0, MAX_DEPTH).astype(i32), (0, npad)),
    ]).reshape(3, NPAD // NCH, NCH).transpose(1, 0, 2)

    ppw = (EPAD - E) // NW
    pad_src = jnp.tile((jnp.arange(ppw, dtype=i32) * 37) % N, (NW, 1))
    pad_dst = jnp.tile(N + jnp.arange(ppw, dtype=i32), (NW, 1))
    src = jnp.concatenate(
        [edge_index[0].astype(i32).reshape(NW, E // NW), pad_src], axis=1)
    dst = jnp.concatenate(
        [edge_index[1].astype(i32).reshape(NW, E // NW), pad_dst], axis=1)
    sd = jnp.stack([src.reshape(TOT_SUP, SB, ECH),
                    dst.reshape(TOT_SUP, SB, ECH)], axis=1)
    batch_p = jnp.pad(batch.astype(i32), (0, npad),
                      constant_values=B).reshape(NPAD, 1)
    zeros_slice = jnp.zeros((SLICE, D), f32)

    h = _embed(idx3, type_emb.astype(f32), attr_emb.astype(f32),
               depth_emb.astype(f32))
    for l in range(L):
        parts = _agg(h, sd, zeros_slice)
        scale = jnp.reshape(1.0 + eps[l], (1,)).astype(f32)
        h = _mlp(l == L - 1, scale, h, parts,
                 W1[l].astype(f32), b1[l].reshape(1, 2 * D).astype(f32),
                 W2[l].astype(f32), b2[l].reshape(1, D).astype(f32),
                 bn_g[l].reshape(1, D).astype(f32),
                 bn_b[l].reshape(1, D).astype(f32))

    g = _pool(h, batch_p)
    preds = _heads(g, head_W.astype(f32),
                   head_b.astype(f32).reshape(S, 1, V))
    return preds


# speedup vs baseline: 1.0197x; 1.0197x over previous
import functools

import jax
import jax.numpy as jnp
from jax import lax
from jax.experimental import pallas as pl
from jax.experimental.pallas import tpu as pltpu
from jax.experimental.pallas import tpu_sc as plsc

N = 10000
E = 320000
D = 128
L = 5
B = 128
V = 5002
S = 5
MAX_DEPTH = 20

NW = 32
NPAD = 10240
NPW = NPAD // NW
NCH = 80
ECH = 128
SB = 8
NSUP = 10
TOT_SUP = NW * NSUP
EPAD = TOT_SUP * SB * ECH
SLICE = NPAD // 16
VPAD = 5120
MB = 2048
PB = 1024


def _worker_id():
    return lax.axis_index("s") * 2 + lax.axis_index("c")


def _embed_chunk_start(ci, idx_hbm, temb_hbm, aemb_hbm, demb_hbm,
                       ib, tr, ar, dr, sem):
    pltpu.sync_copy(idx_hbm.at[ci], ib)
    pltpu.async_copy(temb_hbm.at[ib.at[0]], tr, sem)
    pltpu.async_copy(aemb_hbm.at[ib.at[1]], ar, sem)
    pltpu.async_copy(demb_hbm.at[ib.at[2]], dr, sem)


def _embed_body(idx_hbm, temb_hbm, aemb_hbm, demb_hbm, out_hbm,
                ib0, ib1, tr0, tr1, ar0, ar1, dr0, dr1, sem0, sem1):
    wid = _worker_id()
    base0 = wid * NPW
    nch = NPW // NCH
    ci0 = wid * nch
    ibs = (ib0, ib1)
    trs = (tr0, tr1)
    ars = (ar0, ar1)
    drs = (dr0, dr1)
    sems = (sem0, sem1)
    for p in range(2):
        _embed_chunk_start(ci0 + p, idx_hbm, temb_hbm, aemb_hbm,
                           demb_hbm, ibs[p], trs[p], ars[p], drs[p], sems[p])

    @pl.loop(0, nch, step=2)
    def _chunk(c):
        for p in range(2):
            cc = c + p
            base = pl.multiple_of(base0 + cc * NCH, NCH)
            for r in (trs[p], ars[p], drs[p]):
                pltpu.make_async_copy(temb_hbm.at[ibs[p].at[0]], r,
                                      sems[p]).wait()

            @pl.loop(0, NCH)
            def _row(r):
                for j in range(D // 16):
                    sl = pl.ds(j * 16, 16)
                    trs[p][r, sl] = (trs[p][r, sl] + ars[p][r, sl]
                                     + drs[p][r, sl])

            pltpu.sync_copy(trs[p], out_hbm.at[pl.ds(base, NCH)])
            nxt = jnp.minimum(ci0 + cc + 2, ci0 + nch - 1)
            _embed_chunk_start(nxt, idx_hbm, temb_hbm, aemb_hbm, demb_hbm,
                               ibs[p], trs[p], ars[p], drs[p], sems[p])

    for p in range(2):
        for r in (trs[p], ars[p], drs[p]):
            pltpu.make_async_copy(temb_hbm.at[ibs[p].at[0]], r,
                                  sems[p]).wait()


def _embed(idx3, temb, aemb, demb):
    mesh = plsc.VectorSubcoreMesh(core_axis_name="c", subcore_axis_name="s")
    call = pl.kernel(
        _embed_body,
        out_type=jax.ShapeDtypeStruct((NPAD, D), jnp.float32),
        mesh=mesh,
        scratch_types=[
            pltpu.VMEM((3, NCH), jnp.int32),
            pltpu.VMEM((3, NCH), jnp.int32),
            pltpu.VMEM((NCH, D), jnp.float32),
            pltpu.VMEM((NCH, D), jnp.float32),
            pltpu.VMEM((NCH, D), jnp.float32),
            pltpu.VMEM((NCH, D), jnp.float32),
            pltpu.VMEM((NCH, D), jnp.float32),
            pltpu.VMEM((NCH, D), jnp.float32),
            pltpu.SemaphoreType.DMA,
            pltpu.SemaphoreType.DMA,
        ],
    )
    return call(idx3, temb, aemb, demb)


def _edge_pipeline(nsup, sup0, h_hbm, sd_hbm, sdb, isems, rows, sems, agg_sp):
    pltpu.async_copy(sd_hbm.at[sup0], sdb[0], isems[0])
    pltpu.async_copy(sd_hbm.at[sup0 + 1], sdb[1], isems[1])

    @pl.loop(0, nsup, step=2)
    def _super(so):
        for p in range(2):
            s = sup0 + so + p
            pltpu.make_async_copy(sd_hbm.at[s], sdb[p], isems[p]).wait()
            pltpu.async_copy(h_hbm.at[sdb[p].at[0, 0]], rows[0], sems[0])
            pltpu.async_copy(h_hbm.at[sdb[p].at[0, 1]], rows[1], sems[1])
            for j in range(SB):
                b = j % 2
                pltpu.make_async_copy(h_hbm.at[sdb[p].at[0, j]], rows[b],
                                      sems[b]).wait()
                pltpu.sync_copy(rows[b], agg_sp.at[sdb[p].at[1, j]],
                                add=True)
                if j < SB - 2:
                    pltpu.async_copy(h_hbm.at[sdb[p].at[0, j + 2]],
                                     rows[b], sems[b])
            nxt = jnp.minimum(s + 2, sup0 + nsup - 1)
            pltpu.async_copy(sd_hbm.at[nxt], sdb[p], isems[p])

    for p in range(2):
        pltpu.make_async_copy(sd_hbm.at[sup0], sdb[p], isems[p]).wait()


def _agg_body(h_hbm, sd_hbm, zz_hbm, out_hbm,
              sd0, sd1, rows0, rows1, agg_sp, isem0, isem1, sem0, sem1):
    cid = lax.axis_index("c")
    sid = lax.axis_index("s")
    sdb = (sd0, sd1)
    isems = (isem0, isem1)
    rows = (rows0, rows1)
    sems = (sem0, sem1)
    with jax.named_scope("agg_zero"):
        pltpu.sync_copy(zz_hbm, agg_sp.at[pl.ds(sid * SLICE, SLICE)])
        plsc.subcore_barrier()

    with jax.named_scope("agg_edges"):
        wid = sid * 2 + cid
        _edge_pipeline(NSUP, wid * NSUP, h_hbm, sd_hbm, sdb, isems,
                       rows, sems, agg_sp)

    with jax.named_scope("agg_drain"):
        plsc.subcore_barrier()
        nd = SLICE // ECH
        for k in range(nd):
            b = k % 2
            base = sid * SLICE + k * ECH
            if k >= 2:
                pltpu.make_async_copy(
                    rows[b], out_hbm.at[cid, pl.ds(base, ECH)],
                    sems[b]).wait()
            pltpu.sync_copy(agg_sp.at[pl.ds(base, ECH)], rows[b])
            pltpu.async_copy(rows[b], out_hbm.at[cid, pl.ds(base, ECH)],
                             sems[b])
        for b in range(2):
            pltpu.make_async_copy(
                rows[b], out_hbm.at[cid, pl.ds(sid * SLICE, ECH)],
                sems[b]).wait()


def _agg(h, sd, zeros_slice):
    mesh = plsc.VectorSubcoreMesh(core_axis_name="c", subcore_axis_name="s")
    call = pl.kernel(
        _agg_body,
        out_type=jax.ShapeDtypeStruct((2, NPAD, D), jnp.float32),
        mesh=mesh,
        scratch_types=[
            pltpu.VMEM((2, SB, ECH), jnp.int32),
            pltpu.VMEM((2, SB, ECH), jnp.int32),
            pltpu.VMEM((ECH, D), jnp.float32),
            pltpu.VMEM((ECH, D), jnp.float32),
            pltpu.VMEM_SHARED((NPAD, D), jnp.float32),
            pltpu.SemaphoreType.DMA,
            pltpu.SemaphoreType.DMA,
            pltpu.SemaphoreType.DMA,
            pltpu.SemaphoreType.DMA,
        ],
    )
    return call(h, sd, zeros_slice)


def _mlp_body(scale_sr, h_ref, a_ref, w1_ref, b1_ref, w2_ref, b2_ref,
              g_ref, bb_ref, o_ref, *, last):
    x = h_ref[...] * scale_sr[0] + a_ref[0] + a_ref[1]
    y = jnp.dot(x, w1_ref[...], preferred_element_type=jnp.float32)
    y = jnp.maximum(y + b1_ref[...], 0.0)
    z = jnp.dot(y, w2_ref[...], preferred_element_type=jnp.float32)
    z = (z + b2_ref[...]) * g_ref[...] + bb_ref[...]
    if not last:
        z = jnp.maximum(z, 0.0)
    o_ref[...] = z


def _mlp(last, scale, h, parts, w1, b1, w2, b2, bng, bnb):
    call = pl.pallas_call(
        functools.partial(_mlp_body, last=last),
        grid=(NPAD // MB,),
        in_specs=[
            pl.BlockSpec(memory_space=pltpu.SMEM),
            pl.BlockSpec((MB, D), lambda i: (i, 0)),
            pl.BlockSpec((2, MB, D), lambda i: (0, i, 0)),
            pl.BlockSpec((D, 2 * D), lambda i: (0, 0)),
            pl.BlockSpec((1, 2 * D), lambda i: (0, 0)),
            pl.BlockSpec((2 * D, D), lambda i: (0, 0)),
            pl.BlockSpec((1, D), lambda i: (0, 0)),
            pl.BlockSpec((1, D), lambda i: (0, 0)),
            pl.BlockSpec((1, D), lambda i: (0, 0)),
        ],
        out_specs=pl.BlockSpec((MB, D), lambda i: (i, 0)),
        out_shape=jax.ShapeDtypeStruct((NPAD, D), jnp.float32),
    )
    return call(scale, h, parts, w1, b1, w2, b2, bng, bnb)


def _pool_body(h_ref, batch_ref, o_ref):
    def step(i, acc):
        sums, cnts = acc
        hb = h_ref[pl.ds(i * PB, PB), :]
        bb = batch_ref[pl.ds(i * PB, PB), :]
        hb = jnp.where(bb < B, hb, 0.0)
        oh = (bb == lax.broadcasted_iota(jnp.int32, (1, B), 1))
        oh = oh.astype(jnp.float32)
        sums = sums + lax.dot_general(oh, hb, (((0,), (0,)), ((), ())),
                                      preferred_element_type=jnp.float32)
        cnts = cnts + jnp.sum(oh, axis=0)
        return sums, cnts

    sums, cnts = lax.fori_loop(
        0, NPAD // PB, step,
        (jnp.zeros((B, D), jnp.float32), jnp.zeros((B,), jnp.float32)))
    o_ref[...] = sums / jnp.maximum(cnts, 1.0)[:, None]


def _pool(h, batch_p):
    call = pl.pallas_call(
        _pool_body,
        out_shape=jax.ShapeDtypeStruct((B, D), jnp.float32),
    )
    return call(h, batch_p)


def _heads_body(g_ref, w_ref, b_ref, o_ref):
    o_ref[0] = jnp.dot(g_ref[...], w_ref[0],
                       preferred_element_type=jnp.float32) + b_ref[0]


def _heads(g, wp, bp):
    call = pl.pallas_call(
        _heads_body,
        grid=(S,),
        in_specs=[
            pl.BlockSpec((B, D), lambda s: (0, 0)),
            pl.BlockSpec((1, D, V), lambda s: (s, 0, 0)),
            pl.BlockSpec((1, 1, V), lambda s: (s, 0, 0)),
        ],
        out_specs=pl.BlockSpec((1, B, V), lambda s: (s, 0, 0)),
        out_shape=jax.ShapeDtypeStruct((S, B, V), jnp.float32),
    )
    return call(g, wp, bp)


def kernel(node_feat, node_depth, edge_index, batch, type_emb, attr_emb,
           depth_emb, eps, W1, b1, W2, b2, bn_g, bn_b, head_W, head_b):
    f32 = jnp.float32
    i32 = jnp.int32
    npad = NPAD - N
    idx3 = jnp.stack([
        jnp.pad(node_feat[:, 0].astype(i32), (0, npad)),
        jnp.pad(node_feat[:, 1].astype(i32), (0, npad)),
        jnp.pad(jnp.clip(node_depth, 0, MAX_DEPTH).astype(i32), (0, npad)),
    ]).reshape(3, NPAD // NCH, NCH).transpose(1, 0, 2)

    ppw = (EPAD - E) // NW
    pad_src = jnp.tile((jnp.arange(ppw, dtype=i32) * 37) % N, (NW, 1))
    pad_dst = jnp.tile(N + jnp.arange(ppw, dtype=i32), (NW, 1))
    src = jnp.concatenate(
        [edge_index[0].astype(i32).reshape(NW, E // NW), pad_src], axis=1)
    dst = jnp.concatenate(
        [edge_index[1].astype(i32).reshape(NW, E // NW), pad_dst], axis=1)
    sd = jnp.stack([src.reshape(TOT_SUP, SB, ECH),
                    dst.reshape(TOT_SUP, SB, ECH)], axis=1)
    batch_p = jnp.pad(batch.astype(i32), (0, npad),
                      constant_values=B).reshape(NPAD, 1)
    zeros_slice = jnp.zeros((SLICE, D), f32)

    h = _embed(idx3, type_emb.astype(f32), attr_emb.astype(f32),
               depth_emb.astype(f32))
    for l in range(L):
        parts = _agg(h, sd, zeros_slice)
        scale = jnp.reshape(1.0 + eps[l], (1,)).astype(f32)
        h = _mlp(l == L - 1, scale, h, parts,
                 W1[l].astype(f32), b1[l].reshape(1, 2 * D).astype(f32),
                 W2[l].astype(f32), b2[l].reshape(1, D).astype(f32),
                 bn_g[l].reshape(1, D).astype(f32),
                 bn_b[l].reshape(1, D).astype(f32))

    g = _pool(h, batch_p)
    preds = _heads(g, head_W.astype(f32),
                   head_b.astype(f32).reshape(S, 1, V))
    return preds
